# Initial kernel scaffold; baseline (speedup 1.0000x reference)
import jax
import jax.numpy as jnp
from jax.experimental import pallas as pl


def kernel(x, Wq0, Wk0, Wrel0, b0, Ws0, Wq1, Wk1, Wrel1, b1, Ws1, Wq2, Wk2, Wrel2, b2, Ws2, edge_index, edge_type):
    raise NotImplementedError("write your pallas kernel here")


# speedup vs baseline: 1.1889x; 1.1889x over previous
import functools

import jax
import jax.numpy as jnp
from jax import lax
from jax.experimental import pallas as pl
from jax.experimental.pallas import tpu as pltpu
from jax.experimental.pallas import tpu_sc as plsc

N = 10000
E = 320000
D = 128
R = 7
DH = 64

NC = 2
NS = 16
NW = NC * NS
EPW = E // NW
C = 80
NCHUNK = EPW // C
NPAD = 10240
ROWS_PER_TILE = NPAD // NS
ZROWS = 128

BN = 1000


def _pre_body(h_ref, wq_ref, wk_ref, wrel_ref, ws_ref, b_ref,
              qk_ref, hw_ref, hsb_ref):
    h = h_ref[...]
    wqk = jnp.concatenate([wq_ref[...], wk_ref[...]], axis=1)
    qk_ref[...] = jnp.dot(h, wqk, preferred_element_type=jnp.float32)
    hsb_ref[...] = (
        jnp.dot(h, ws_ref[...], preferred_element_type=jnp.float32) + b_ref[...]
    )
    for r in range(R):
        hw_ref[r] = jnp.dot(h, wrel_ref[r], preferred_element_type=jnp.float32)


@jax.jit
def _tc_pre(h, wq, wk, wrel, ws, b2):
    return pl.pallas_call(
        _pre_body,
        grid=(N // BN,),
        in_specs=[
            pl.BlockSpec((BN, D), lambda i: (i, 0)),
            pl.BlockSpec((D, DH), lambda i: (0, 0)),
            pl.BlockSpec((D, DH), lambda i: (0, 0)),
            pl.BlockSpec((R, D, D), lambda i: (0, 0, 0)),
            pl.BlockSpec((D, D), lambda i: (0, 0)),
            pl.BlockSpec((1, D), lambda i: (0, 0)),
        ],
        out_specs=[
            pl.BlockSpec((BN, 2 * DH), lambda i: (i, 0)),
            pl.BlockSpec((R, BN, D), lambda i: (0, i, 0)),
            pl.BlockSpec((BN, D), lambda i: (i, 0)),
        ],
        out_shape=[
            jax.ShapeDtypeStruct((N, 2 * DH), jnp.float32),
            jax.ShapeDtypeStruct((R, N, D), jnp.float32),
            jax.ShapeDtypeStruct((N, D), jnp.float32),
        ],
    )(h, wq, wk, wrel, ws, b2)


def _combine_body(acc_ref, hsb_ref, h_ref, gf_ref):
    hv = jnp.maximum(acc_ref[0] + acc_ref[1] + hsb_ref[...], 0.0)
    h_ref[...] = hv

    @pl.when(pl.program_id(0) == 0)
    def _():
        gf_ref[...] = jnp.zeros_like(gf_ref)

    gf_ref[...] += jnp.sum(hv, axis=0, keepdims=True)


@jax.jit
def _tc_combine(accp, hsb):
    return pl.pallas_call(
        _combine_body,
        grid=(N // BN,),
        in_specs=[
            pl.BlockSpec((2, BN, D), lambda i: (0, i, 0)),
            pl.BlockSpec((BN, D), lambda i: (i, 0)),
        ],
        out_specs=[
            pl.BlockSpec((BN, D), lambda i: (i, 0)),
            pl.BlockSpec((1, D), lambda i: (0, 0)),
        ],
        out_shape=[
            jax.ShapeDtypeStruct((N, D), jnp.float32),
            jax.ShapeDtypeStruct((1, D), jnp.float32),
        ],
    )(accp, hsb)


def _edge_body(qk_hbm, hw_hbm, src_hbm, dst_hbm, et_hbm, out_hbm,
               sbuf, dbuf, gbuf, qbuf, kbuf, mbuf, zbuf, acc,
               sem_q, sem_k, sem_m):
    cid = lax.axis_index("c")
    sid = lax.axis_index("s")
    wid = sid * NC + cid

    zv = jnp.zeros((16,), jnp.float32)

    def _zero_row(r, _):
        for j in range(D // 16):
            zbuf[r, pl.ds(j * 16, 16)] = zv
        return 0

    lax.fori_loop(0, ZROWS, _zero_row, 0)
    for p in range(ROWS_PER_TILE // ZROWS):
        pltpu.sync_copy(zbuf, acc.at[pl.ds(sid * ROWS_PER_TILE + p * ZROWS, ZROWS)])
    plsc.subcore_barrier()

    iota16 = lax.iota(jnp.int32, 16)

    def _chunk(ch, _):
        base = wid * EPW + ch * C
        pltpu.sync_copy(src_hbm.at[pl.ds(base, C)], sbuf)
        pltpu.sync_copy(dst_hbm.at[pl.ds(base, C)], dbuf)
        pltpu.sync_copy(et_hbm.at[pl.ds(base, C)], gbuf)

        for g in range(C // 16):
            ev = gbuf[pl.ds(g * 16, 16)]
            sv = sbuf[pl.ds(g * 16, 16)]
            gbuf[pl.ds(g * 16, 16)] = ev * N + sv

        cp_q = pltpu.async_copy(qk_hbm.at[sbuf], qbuf, sem_q)
        cp_k = pltpu.async_copy(qk_hbm.at[dbuf], kbuf, sem_k)
        cp_m = pltpu.async_copy(hw_hbm.at[gbuf], mbuf, sem_m)

        cp_q.wait()
        cp_k.wait()
        cp_m.wait()

        def _group(g, _):
            eidx = iota16 + g * 16

            def _dot(dd, a):
                dvec = jnp.broadcast_to(dd, (16,))
                qv = plsc.load_gather(qbuf, [eidx, dvec])
                kv = plsc.load_gather(kbuf, [eidx, dvec + DH])
                return a + qv * kv

            z = lax.fori_loop(0, DH, _dot, jnp.zeros((16,), jnp.float32))
            z = z * 0.125
            w = 1.0 / (1.0 + jnp.exp(-z))

            def _scale(dd, _):
                dvec = jnp.broadcast_to(dd, (16,))
                col = plsc.load_gather(mbuf, [eidx, dvec])
                plsc.store_scatter(mbuf, [eidx, dvec], col * w)
                return 0

            lax.fori_loop(0, D, _scale, 0)
            return 0

        lax.fori_loop(0, C // 16, _group, 0)

        pltpu.sync_copy(mbuf, acc.at[dbuf], add=True)
        return 0

    lax.fori_loop(0, NCHUNK, _chunk, 0)

    plsc.subcore_barrier()
    pltpu.sync_copy(
        acc.at[pl.ds(sid * ROWS_PER_TILE, ROWS_PER_TILE)],
        out_hbm.at[cid, pl.ds(sid * ROWS_PER_TILE, ROWS_PER_TILE)],
    )


@jax.jit
def _sc_edge(qk, hw_flat, src, dst, et):
    mesh = plsc.VectorSubcoreMesh(core_axis_name="c", subcore_axis_name="s")
    f = functools.partial(
        pl.kernel,
        mesh=mesh,
        compiler_params=pltpu.CompilerParams(needs_layout_passes=False),
        out_type=jax.ShapeDtypeStruct((2, NPAD, D), jnp.float32),
        scratch_types=[
            pltpu.VMEM((C,), jnp.int32),
            pltpu.VMEM((C,), jnp.int32),
            pltpu.VMEM((C,), jnp.int32),
            pltpu.VMEM((C, 2 * DH), jnp.float32),
            pltpu.VMEM((C, 2 * DH), jnp.float32),
            pltpu.VMEM((C, D), jnp.float32),
            pltpu.VMEM((ZROWS, D), jnp.float32),
            pltpu.VMEM_SHARED((NPAD, D), jnp.float32),
            pltpu.SemaphoreType.DMA,
            pltpu.SemaphoreType.DMA,
            pltpu.SemaphoreType.DMA,
        ],
    )(_edge_body)
    return f(qk, hw_flat, src, dst, et)


def kernel(x, Wq0, Wk0, Wrel0, b0, Ws0, Wq1, Wk1, Wrel1, b1, Ws1,
           Wq2, Wk2, Wrel2, b2, Ws2, edge_index, edge_type):
    src = edge_index[0].astype(jnp.int32)
    dst = edge_index[1].astype(jnp.int32)
    et = edge_type.astype(jnp.int32)

    layers = [
        (Wq0, Wk0, Wrel0, b0, Ws0),
        (Wq1, Wk1, Wrel1, b1, Ws1),
        (Wq2, Wk2, Wrel2, b2, Ws2),
    ]
    h = x
    gf = None
    for (Wq, Wk, Wrel, b, Ws) in layers:
        qk, hw, hsb = _tc_pre(h, Wq, Wk, Wrel.reshape(R, D, D), Ws,
                              b.reshape(1, D))
        accp = _sc_edge(qk, hw.reshape(R * N, D), src, dst, et)
        h, gf = _tc_combine(accp, hsb)
    return gf, h
